# Initial kernel scaffold; baseline (speedup 1.0000x reference)
import jax
import jax.numpy as jnp
from jax.experimental import pallas as pl


def kernel(features, coords, Wf, bf, Wq, bq, Wk, bk, Wv, bv, Wo, bo, pos_emb, ln_gamma, ln_beta):
    raise NotImplementedError("write your pallas kernel here")


# speedup vs baseline: 2.3248x; 2.3248x over previous
import functools

import jax
import jax.numpy as jnp
import numpy as np
from jax import lax
from jax.experimental import pallas as pl
from jax.experimental.pallas import tpu as pltpu
from jax.experimental.pallas import tpu_sc as plsc

_B, _N, _D = 2, 4096, 128
_NX, _NY, _NT = 10, 10, 200
_M = _NX * _NY * _NT
_R = _B * _M
_K = 27
_P = _B * _N
_NCORE, _NSUB, _NW = 2, 16, 32
_PPT = _P // _NSUB
_CHUNK = 128
_NCH = _PPT // _CHUNK
_STRIPE = _R // _NSUB
_GROWS = _P * _K // _NW
_GCH = _GROWS // _CHUNK
_PBLK = 512
_NPB = _P // _PBLK
_INV = 1.0 / float(np.sqrt(_D))

_xs = np.arange(-1, 2)
_gz, _gy, _gx = np.meshgrid(_xs, _xs, _xs, indexing="ij")
_OFF = np.stack([_gx.ravel(), _gy.ravel(), _gz.ravel()], axis=-1).astype(np.int32)


def _feat_body(x_ref, wf_ref, bf_ref, o_ref):
    o_ref[...] = (
        jnp.dot(x_ref[...], wf_ref[...], preferred_element_type=jnp.float32)
        + bf_ref[...]
    )


def _feat_call(x, wf, bf):
    return pl.pallas_call(
        _feat_body,
        grid=(_NPB,),
        in_specs=[
            pl.BlockSpec((_PBLK, _D), lambda i: (i, 0)),
            pl.BlockSpec((_D, _D), lambda i: (0, 0)),
            pl.BlockSpec((1, _D), lambda i: (0, 0)),
        ],
        out_specs=pl.BlockSpec((_PBLK, _D), lambda i: (i, 0)),
        out_shape=jax.ShapeDtypeStruct((_P, _D), jnp.float32),
    )(x, wf, bf)


def _sc_mesh():
    return plsc.VectorSubcoreMesh(
        core_axis_name="c", subcore_axis_name="s",
        num_cores=_NCORE, num_subcores=_NSUB,
    )


def _scatter_body(feat_hbm, gid_hbm, sums_hbm, cnt_hbm,
                  slab, zbuf, gidv, src, ones):
    c = lax.axis_index("c")
    s = lax.axis_index("s")
    z16 = jnp.zeros((16,), jnp.float32)
    one16 = jnp.where(lax.iota(jnp.int32, 16) == 0, 1.0, 0.0).astype(jnp.float32)

    def fill_z(i, carry):
        zbuf[i] = z16
        return carry
    lax.fori_loop(0, _STRIPE, fill_z, 0)

    def fill_o(i, carry):
        ones[i] = one16
        return carry
    lax.fori_loop(0, _CHUNK, fill_o, 0)

    pltpu.sync_copy(gid_hbm.at[s], gidv)

    def one_pass(col_off, count_pass):
        pltpu.sync_copy(zbuf, slab.at[pl.ds(s * _STRIPE, _STRIPE)])
        plsc.subcore_barrier()
        for ch in range(_NCH):
            if not count_pass:
                pltpu.sync_copy(
                    feat_hbm.at[pl.ds(s * _PPT + ch * _CHUNK, _CHUNK),
                                pl.ds(col_off, 16)],
                    src,
                )
                pltpu.sync_copy(src, slab.at[gidv.at[ch]], add=True)
            else:
                pltpu.sync_copy(ones, slab.at[gidv.at[ch]], add=True)
        plsc.subcore_barrier()
        if not count_pass:
            pltpu.sync_copy(
                slab.at[pl.ds(s * _STRIPE, _STRIPE)],
                sums_hbm.at[pl.ds(s * _STRIPE, _STRIPE), pl.ds(col_off, 16)],
            )
        else:
            pltpu.sync_copy(
                slab.at[pl.ds(s * _STRIPE, _STRIPE)],
                cnt_hbm.at[pl.ds(s * _STRIPE, _STRIPE)],
            )
        plsc.subcore_barrier()

    for jj in range(4):
        one_pass((c * 4 + jj) * 16, False)

    @pl.when(c == 0)
    def _():
        one_pass(0, True)


def _scatter_call(feat, gid3):
    k = pl.kernel(
        _scatter_body,
        out_type=(
            jax.ShapeDtypeStruct((_R, _D), jnp.float32),
            jax.ShapeDtypeStruct((_R, 16), jnp.float32),
        ),
        mesh=_sc_mesh(),
        compiler_params=pltpu.CompilerParams(use_tc_tiling_on_sc=False),
        scratch_types=[
            pltpu.VMEM_SHARED((_R, 16), jnp.float32),
            pltpu.VMEM((_STRIPE, 16), jnp.float32),
            pltpu.VMEM((_NCH, _CHUNK), jnp.int32),
            pltpu.VMEM((_CHUNK, 16), jnp.float32),
            pltpu.VMEM((_CHUNK, 16), jnp.float32),
        ],
    )
    return k(feat, gid3)


def _norm_body(sums_ref, cnt_ref, o_ref):
    c = cnt_ref[:, 0:1]
    o_ref[...] = sums_ref[...] * (1.0 / jnp.maximum(c, 1.0))


def _norm_call(sums, cnt16):
    blk = 2000
    return pl.pallas_call(
        _norm_body,
        grid=(_R // blk,),
        in_specs=[
            pl.BlockSpec((blk, _D), lambda i: (i, 0)),
            pl.BlockSpec((blk, 16), lambda i: (i, 0)),
        ],
        out_specs=pl.BlockSpec((blk, _D), lambda i: (i, 0)),
        out_shape=jax.ShapeDtypeStruct((_R, _D), jnp.float32),
    )(sums, cnt16)


def _gather_body(table_hbm, sel_hbm, out_hbm, idxv, buf, sem):
    c = lax.axis_index("c")
    s = lax.axis_index("s")
    wid = s * _NCORE + c
    base = wid * _GROWS
    pltpu.sync_copy(sel_hbm.at[pl.ds(base, _GROWS)], idxv)

    def body(ch, carry):
        off = pl.multiple_of(ch * _CHUNK, _CHUNK)
        pltpu.async_copy(
            table_hbm.at[idxv.at[pl.ds(off, _CHUNK)]], buf, sem
        ).wait()
        pltpu.sync_copy(buf, out_hbm.at[pl.ds(base + off, _CHUNK)])
        return carry

    lax.fori_loop(0, _GCH, body, 0)


def _gather_call(table, sel):
    k = pl.kernel(
        _gather_body,
        out_type=jax.ShapeDtypeStruct((_P * _K, _D), jnp.float32),
        mesh=_sc_mesh(),
        compiler_params=pltpu.CompilerParams(use_tc_tiling_on_sc=False),
        scratch_types=[
            pltpu.VMEM((_GROWS,), jnp.int32),
            pltpu.VMEM((_CHUNK, _D), jnp.float32),
            pltpu.SemaphoreType.DMA,
        ],
    )
    return k(table, sel)


def _attn_body(gth_ref, feat_ref, vb_ref, wq_ref, bq_ref, wkt_ref, post_ref,
               pos_ref, bk_ref, wv_ref, bv_ref, wo_ref, bo_ref, gam_ref,
               bet_ref, o_ref):
    f = feat_ref[...]
    q = jnp.dot(f, wq_ref[...], preferred_element_type=jnp.float32) + bq_ref[...]
    qk = jnp.dot(q, wkt_ref[...], preferred_element_type=jnp.float32) * _INV
    g = gth_ref[...]
    s = jnp.sum(qk[:, None, :] * g, axis=-1)
    s = s + jnp.dot(qk, post_ref[...], preferred_element_type=jnp.float32)
    s = s + jnp.sum(q * bk_ref[...], axis=-1, keepdims=True) * _INV
    s = s + vb_ref[...]
    m = jnp.max(s, axis=-1, keepdims=True)
    e = jnp.exp(s - m)
    a = e / jnp.sum(e, axis=-1, keepdims=True)
    pooled = jnp.sum(a[:, :, None] * g, axis=1)
    pooled = pooled + jnp.dot(a, pos_ref[...], preferred_element_type=jnp.float32)
    out = jnp.dot(pooled, wv_ref[...], preferred_element_type=jnp.float32) + bv_ref[...]
    out = jnp.dot(out, wo_ref[...], preferred_element_type=jnp.float32) + bo_ref[...]
    res = f + out
    mu = jnp.mean(res, axis=-1, keepdims=True)
    var = jnp.mean((res - mu) ** 2, axis=-1, keepdims=True)
    o_ref[...] = (res - mu) * lax.rsqrt(var + 1e-5) * gam_ref[...] + bet_ref[...]


def _attn_call(gth3, feat, vb, wq, bq, wkt, post, pos, bk, wv, bv, wo, bo,
               gam, bet):
    full = lambda r, c: pl.BlockSpec((r, c), lambda i: (0, 0))
    return pl.pallas_call(
        _attn_body,
        grid=(_NPB,),
        in_specs=[
            pl.BlockSpec((_PBLK, _K, _D), lambda i: (i, 0, 0)),
            pl.BlockSpec((_PBLK, _D), lambda i: (i, 0)),
            pl.BlockSpec((_PBLK, _K), lambda i: (i, 0)),
            full(_D, _D), full(1, _D), full(_D, _D), full(_D, _K),
            full(_K, _D), full(1, _D), full(_D, _D), full(1, _D),
            full(_D, _D), full(1, _D), full(1, _D), full(1, _D),
        ],
        out_specs=pl.BlockSpec((_PBLK, _D), lambda i: (i, 0)),
        out_shape=jax.ShapeDtypeStruct((_P, _D), jnp.float32),
    )(gth3, feat, vb, wq, bq, wkt, post, pos, bk, wv, bv, wo, bo, gam, bet)


def kernel(features, coords, Wf, bf, Wq, bq, Wk, bk, Wv, bv, Wo, bo,
           pos_emb, ln_gamma, ln_beta):
    off = jnp.asarray(_OFF)
    x = features.reshape(_P, _D)

    ix = (jnp.clip(coords[..., 0], 0.0, 1.0) * (_NX - 1)).astype(jnp.int32)
    iy = (jnp.clip(coords[..., 1], 0.0, 1.0) * (_NY - 1)).astype(jnp.int32)
    it = (jnp.clip(coords[..., 2], 0.0, 1.0) * (_NT - 1)).astype(jnp.int32)
    vidx = jnp.stack([ix, iy, it], axis=-1)
    lin = vidx[..., 0] + vidx[..., 1] * _NX + vidx[..., 2] * (_NX * _NY)
    bbase = (jnp.arange(_B, dtype=jnp.int32) * _M)[:, None]
    gid = (lin + bbase).reshape(-1)
    gid3 = gid.reshape(_NSUB, _NCH, _CHUNK)
    nbr = vidx[:, :, None, :] + off[None, None, :, :]
    valid = ((nbr[..., 0] >= 0) & (nbr[..., 0] < _NX)
             & (nbr[..., 1] >= 0) & (nbr[..., 1] < _NY)
             & (nbr[..., 2] >= 0) & (nbr[..., 2] < _NT))
    nlin = (jnp.clip(nbr[..., 0], 0, _NX - 1)
            + jnp.clip(nbr[..., 1], 0, _NY - 1) * _NX
            + jnp.clip(nbr[..., 2], 0, _NT - 1) * (_NX * _NY))
    sel = (nlin + bbase[:, :, None]).reshape(-1)
    vb = jnp.where(valid, 0.0, -1e9).astype(jnp.float32).reshape(_P, _K)

    feat = _feat_call(x, Wf, bf.reshape(1, _D))
    sums, cnt16 = _scatter_call(feat, gid3)
    table = _norm_call(sums, cnt16)
    gth = _gather_call(table, sel)
    y = _attn_call(
        gth.reshape(_P, _K, _D), feat, vb,
        Wq, bq.reshape(1, _D), Wk.T, pos_emb.T, pos_emb,
        bk.reshape(1, _D), Wv, bv.reshape(1, _D), Wo, bo.reshape(1, _D),
        ln_gamma.reshape(1, _D), ln_beta.reshape(1, _D),
    )
    return y.reshape(_B, _N, _D)
